# baseline (speedup 1.0000x reference)
import functools

import jax
import jax.numpy as jnp
from jax import lax
from jax.experimental import pallas as pl
from jax.experimental.pallas import tpu as pltpu
from jax.experimental.pallas import tpu_sc as plsc

NC = 2
NS = 16
L = 16
NW = NC * NS

B = 16384
D = 32
SUB = 8
PLANES = D // SUB
PW = B // NW
GROUPS = PW // L
EPB = SUB
BIAS_SLOT = L * L

_mesh = plsc.VectorSubcoreMesh(core_axis_name="c", subcore_axis_name="s")


@functools.partial(
    pl.kernel,
    out_type=jax.ShapeDtypeStruct((B,), jnp.float32),
    mesh=_mesh,
    scratch_types=[
        pltpu.VMEM((PW,), jnp.int32),
        pltpu.VMEM((PW,), jnp.int32),
        pltpu.VMEM((2, 2, PLANES, SUB, L * EPB), jnp.float32),
        pltpu.VMEM((2, 2, PLANES, SUB, L * EPB), jnp.float32),
        pltpu.VMEM((2 * BIAS_SLOT,), jnp.float32),
        pltpu.VMEM((2 * BIAS_SLOT,), jnp.float32),
        pltpu.VMEM((L,), jnp.float32),
        pltpu.VMEM((PW,), jnp.float32),
        pltpu.SemaphoreType.DMA,
        pltpu.SemaphoreType.DMA,
    ],
    compiler_params=pltpu.CompilerParams(needs_layout_passes=False),
)
def _sc_predict(uidx_hbm, iidx_hbm, uembd_hbm, iembd_hbm, ubias_hbm,
                ibias_hbm, oab_hbm, out_hbm,
                uidx_v, iidx_v, useg_v, iseg_v, ubias_v, ibias_v,
                oab_v, out_v, sem0, sem1):
    wid = lax.axis_index("s") * NC + lax.axis_index("c")
    base = wid * PW

    pltpu.sync_copy(uidx_hbm.at[pl.ds(base, PW)], uidx_v)
    pltpu.sync_copy(iidx_hbm.at[pl.ds(base, PW)], iidx_v)
    pltpu.sync_copy(oab_hbm, oab_v)

    oab = oab_v[...]
    lanes = lax.iota(jnp.int32, L)

    def issue(gi, slot, sem):
        j0 = gi * L
        uvec = uidx_v[pl.ds(j0, L)]
        ivec = iidx_v[pl.ds(j0, L)]
        for t in range(L):
            blk_t = t // EPB
            e = t % EPB
            vu16 = pl.multiple_of((uvec[t] >> 4) << 4, L)
            vi16 = pl.multiple_of((ivec[t] >> 4) << 4, L)
            de = pl.multiple_of(e * L, L)
            dt = pl.multiple_of(slot * BIAS_SLOT + t * L, L)
            pltpu.async_copy(
                uembd_hbm.at[:, :, pl.ds(vu16, L)],
                useg_v.at[slot, blk_t, :, :, pl.ds(de, L)], sem)
            pltpu.async_copy(
                iembd_hbm.at[:, :, pl.ds(vi16, L)],
                iseg_v.at[slot, blk_t, :, :, pl.ds(de, L)], sem)
            pltpu.async_copy(
                ubias_hbm.at[pl.ds(vu16, L)], ubias_v.at[pl.ds(dt, L)], sem)
            pltpu.async_copy(
                ibias_hbm.at[pl.ds(vi16, L)], ibias_v.at[pl.ds(dt, L)], sem)

    def drain(slot, sem):
        for t in range(L):
            blk_t = t // EPB
            e = t % EPB
            de = pl.multiple_of(e * L, L)
            dt = pl.multiple_of(slot * BIAS_SLOT + t * L, L)
            pltpu.make_async_copy(
                uembd_hbm.at[:, :, pl.ds(0, L)],
                useg_v.at[slot, blk_t, :, :, pl.ds(de, L)], sem).wait()
            pltpu.make_async_copy(
                iembd_hbm.at[:, :, pl.ds(0, L)],
                iseg_v.at[slot, blk_t, :, :, pl.ds(de, L)], sem).wait()
            pltpu.make_async_copy(
                ubias_hbm.at[pl.ds(0, L)], ubias_v.at[pl.ds(dt, L)], sem).wait()
            pltpu.make_async_copy(
                ibias_hbm.at[pl.ds(0, L)], ibias_v.at[pl.ds(dt, L)], sem).wait()

    def compute(gi, slot):
        j0 = gi * L
        uvec = uidx_v[pl.ds(j0, L)]
        ivec = iidx_v[pl.ds(j0, L)]
        blk = lanes >> 3
        slot_f = jnp.zeros((L,), jnp.int32) + slot
        minor_u = ((lanes & (EPB - 1)) << 4) + (uvec & (L - 1))
        minor_i = ((lanes & (EPB - 1)) << 4) + (ivec & (L - 1))
        uboff = slot * BIAS_SLOT + lanes * L + (uvec & (L - 1))
        iboff = slot * BIAS_SLOT + lanes * L + (ivec & (L - 1))
        acc = (plsc.load_gather(ubias_v, [uboff])
               + plsc.load_gather(ibias_v, [iboff]) + oab)
        for a in range(PLANES):
            af = jnp.full((L,), a, jnp.int32)
            for r in range(SUB):
                rf = jnp.full((L,), r, jnp.int32)
                acc = acc + (
                    plsc.load_gather(useg_v, [slot_f, blk, af, rf, minor_u])
                    * plsc.load_gather(iseg_v, [slot_f, blk, af, rf, minor_i]))
        out_v[pl.ds(j0, L)] = acc

    issue(0, 0, sem0)

    def pipe2(hi, carry):
        g_even = hi * 2
        g_odd = hi * 2 + 1
        issue(g_odd, 1, sem1)
        drain(0, sem0)
        compute(g_even, 0)

        @pl.when(hi < GROUPS // 2 - 1)
        def _():
            issue(g_even + 2, 0, sem0)

        drain(1, sem1)
        compute(g_odd, 1)
        return carry

    lax.fori_loop(0, GROUPS // 2, pipe2, 0)

    pltpu.sync_copy(out_v, out_hbm.at[pl.ds(base, PW)])


def kernel(userIdx, itemIdx, uEmbd, iEmbd, uBias, iBias, overAllBias):
    uidx = userIdx.astype(jnp.int32)
    iidx = itemIdx.astype(jnp.int32)
    ut = uEmbd.T.reshape(PLANES, SUB, uEmbd.shape[0])
    it = iEmbd.T.reshape(PLANES, SUB, iEmbd.shape[0])
    ubias = lax.reshape(uBias, (uBias.shape[0],), dimensions=(1, 0))
    ibias = lax.reshape(iBias, (iBias.shape[0],), dimensions=(1, 0))
    oab = jnp.broadcast_to(overAllBias.astype(jnp.float32), (L,))
    return _sc_predict(uidx, iidx, ut, it, ubias, ibias, oab)


# speedup vs baseline: 1.0389x; 1.0389x over previous
import functools

import jax
import jax.numpy as jnp
from jax import lax
from jax.experimental import pallas as pl
from jax.experimental.pallas import tpu as pltpu
from jax.experimental.pallas import tpu_sc as plsc

NC = 2
NS = 16
L = 16
NW = NC * NS

B = 16384
D = 32
SUB = 8
PLANES = D // SUB
PW = B // NW
GROUPS = PW // L
EPB = SUB
BIAS_SLOT = L * L

_mesh = plsc.VectorSubcoreMesh(core_axis_name="c", subcore_axis_name="s")


@functools.partial(
    pl.kernel,
    out_type=jax.ShapeDtypeStruct((B,), jnp.float32),
    mesh=_mesh,
    scratch_types=[
        pltpu.VMEM((PW,), jnp.int32),
        pltpu.VMEM((PW,), jnp.int32),
        pltpu.VMEM((2, 2, PLANES, SUB, L * EPB), jnp.float32),
        pltpu.VMEM((2, 2, PLANES, SUB, L * EPB), jnp.float32),
        pltpu.VMEM((2 * BIAS_SLOT,), jnp.float32),
        pltpu.VMEM((2 * BIAS_SLOT,), jnp.float32),
        pltpu.VMEM((L,), jnp.float32),
        pltpu.VMEM((PW,), jnp.float32),
        pltpu.SemaphoreType.DMA,
        pltpu.SemaphoreType.DMA,
    ],
    compiler_params=pltpu.CompilerParams(needs_layout_passes=False),
)
def _sc_predict(uidx_hbm, iidx_hbm, uembd_hbm, iembd_hbm, ubias_hbm,
                ibias_hbm, oab_hbm, out_hbm,
                uidx_v, iidx_v, useg_v, iseg_v, ubias_v, ibias_v,
                oab_v, out_v, sem0, sem1):
    wid = lax.axis_index("s") * NC + lax.axis_index("c")
    base = wid * PW

    pltpu.sync_copy(uidx_hbm.at[pl.ds(base, PW)], uidx_v)
    pltpu.sync_copy(iidx_hbm.at[pl.ds(base, PW)], iidx_v)
    pltpu.sync_copy(oab_hbm, oab_v)

    oab = oab_v[...]
    lanes = lax.iota(jnp.int32, L)

    def issue(gi, slot, sem):
        j0 = gi * L
        uvec = uidx_v[pl.ds(j0, L)]
        ivec = iidx_v[pl.ds(j0, L)]
        for t in range(L):
            blk_t = t // EPB
            e = t % EPB
            vu16 = pl.multiple_of((uvec[t] >> 4) << 4, L)
            vi16 = pl.multiple_of((ivec[t] >> 4) << 4, L)
            de = pl.multiple_of(e * L, L)
            dt = pl.multiple_of(slot * BIAS_SLOT + t * L, L)
            pltpu.async_copy(
                uembd_hbm.at[:, :, pl.ds(vu16, L)],
                useg_v.at[slot, blk_t, :, :, pl.ds(de, L)], sem)
            pltpu.async_copy(
                iembd_hbm.at[:, :, pl.ds(vi16, L)],
                iseg_v.at[slot, blk_t, :, :, pl.ds(de, L)], sem)
            pltpu.async_copy(
                ubias_hbm.at[pl.ds(vu16, L)], ubias_v.at[pl.ds(dt, L)], sem)
            pltpu.async_copy(
                ibias_hbm.at[pl.ds(vi16, L)], ibias_v.at[pl.ds(dt, L)], sem)

    def drain(slot, sem):
        for blk_t in range(2):
            pltpu.make_async_copy(
                uembd_hbm.at[:, :, pl.ds(0, L * EPB)],
                useg_v.at[slot, blk_t], sem).wait()
            pltpu.make_async_copy(
                iembd_hbm.at[:, :, pl.ds(0, L * EPB)],
                iseg_v.at[slot, blk_t], sem).wait()
        dslot = pl.multiple_of(slot * BIAS_SLOT, L)
        pltpu.make_async_copy(
            ubias_hbm.at[pl.ds(0, BIAS_SLOT)],
            ubias_v.at[pl.ds(dslot, BIAS_SLOT)], sem).wait()
        pltpu.make_async_copy(
            ibias_hbm.at[pl.ds(0, BIAS_SLOT)],
            ibias_v.at[pl.ds(dslot, BIAS_SLOT)], sem).wait()

    def compute(gi, slot):
        j0 = gi * L
        uvec = uidx_v[pl.ds(j0, L)]
        ivec = iidx_v[pl.ds(j0, L)]
        blk = lanes >> 3
        slot_f = jnp.zeros((L,), jnp.int32) + slot
        minor_u = ((lanes & (EPB - 1)) << 4) + (uvec & (L - 1))
        minor_i = ((lanes & (EPB - 1)) << 4) + (ivec & (L - 1))
        uboff = slot * BIAS_SLOT + lanes * L + (uvec & (L - 1))
        iboff = slot * BIAS_SLOT + lanes * L + (ivec & (L - 1))
        acc = (plsc.load_gather(ubias_v, [uboff])
               + plsc.load_gather(ibias_v, [iboff]) + oab)
        for a in range(PLANES):
            af = jnp.full((L,), a, jnp.int32)
            for r in range(SUB):
                rf = jnp.full((L,), r, jnp.int32)
                acc = acc + (
                    plsc.load_gather(useg_v, [slot_f, blk, af, rf, minor_u])
                    * plsc.load_gather(iseg_v, [slot_f, blk, af, rf, minor_i]))
        out_v[pl.ds(j0, L)] = acc

    issue(0, 0, sem0)

    def pipe2(hi, carry):
        g_even = hi * 2
        g_odd = hi * 2 + 1
        issue(g_odd, 1, sem1)
        drain(0, sem0)
        compute(g_even, 0)

        @pl.when(hi < GROUPS // 2 - 1)
        def _():
            issue(g_even + 2, 0, sem0)

        drain(1, sem1)
        compute(g_odd, 1)
        return carry

    lax.fori_loop(0, GROUPS // 2, pipe2, 0)

    pltpu.sync_copy(out_v, out_hbm.at[pl.ds(base, PW)])


def kernel(userIdx, itemIdx, uEmbd, iEmbd, uBias, iBias, overAllBias):
    uidx = userIdx.astype(jnp.int32)
    iidx = itemIdx.astype(jnp.int32)
    ut = uEmbd.T.reshape(PLANES, SUB, uEmbd.shape[0])
    it = iEmbd.T.reshape(PLANES, SUB, iEmbd.shape[0])
    ubias = lax.reshape(uBias, (uBias.shape[0],), dimensions=(1, 0))
    ibias = lax.reshape(iBias, (iBias.shape[0],), dimensions=(1, 0))
    oab = jnp.broadcast_to(overAllBias.astype(jnp.float32), (L,))
    return _sc_predict(uidx, iidx, ut, it, ubias, ibias, oab)


# speedup vs baseline: 1.0593x; 1.0197x over previous
import functools

import jax
import jax.numpy as jnp
from jax import lax
from jax.experimental import pallas as pl
from jax.experimental.pallas import tpu as pltpu
from jax.experimental.pallas import tpu_sc as plsc

NC = 2
NS = 16
L = 16
NW = NC * NS

B = 16384
D = 32
SUB = 8
PLANES = D // SUB
PW = B // NW
GROUPS = PW // L
EPB = SUB
BIAS_SLOT = L * L

_mesh = plsc.VectorSubcoreMesh(core_axis_name="c", subcore_axis_name="s")


@functools.partial(
    pl.kernel,
    out_type=jax.ShapeDtypeStruct((B,), jnp.float32),
    mesh=_mesh,
    scratch_types=[
        pltpu.VMEM((PW,), jnp.int32),
        pltpu.VMEM((PW,), jnp.int32),
        pltpu.VMEM((2, 2, PLANES, SUB, L * EPB), jnp.float32),
        pltpu.VMEM((2, 2, PLANES, SUB, L * EPB), jnp.float32),
        pltpu.VMEM((2 * BIAS_SLOT,), jnp.float32),
        pltpu.VMEM((2 * BIAS_SLOT,), jnp.float32),
        pltpu.VMEM((L,), jnp.float32),
        pltpu.VMEM((PW,), jnp.float32),
        pltpu.SemaphoreType.DMA,
        pltpu.SemaphoreType.DMA,
    ],
    compiler_params=pltpu.CompilerParams(needs_layout_passes=False),
)
def _sc_predict(uidx_hbm, iidx_hbm, uembd_hbm, iembd_hbm, ubias_hbm,
                ibias_hbm, oab_hbm, out_hbm,
                uidx_v, iidx_v, useg_v, iseg_v, ubias_v, ibias_v,
                oab_v, out_v, sem0, sem1):
    wid = lax.axis_index("s") * NC + lax.axis_index("c")
    base = wid * PW

    pltpu.sync_copy(uidx_hbm.at[pl.ds(base, PW)], uidx_v)
    pltpu.sync_copy(iidx_hbm.at[pl.ds(base, PW)], iidx_v)
    pltpu.sync_copy(oab_hbm, oab_v)

    oab = oab_v[...]
    lanes = lax.iota(jnp.int32, L)

    def issue(gi, slot, sem):
        j0 = gi * L
        uvec = uidx_v[pl.ds(j0, L)]
        ivec = iidx_v[pl.ds(j0, L)]
        for t in range(L):
            blk_t = t // EPB
            e = t % EPB
            vu16 = pl.multiple_of((uvec[t] >> 4) << 4, L)
            vi16 = pl.multiple_of((ivec[t] >> 4) << 4, L)
            de = pl.multiple_of(e * L, L)
            dt = pl.multiple_of(slot * BIAS_SLOT + t * L, L)
            pltpu.async_copy(
                uembd_hbm.at[:, :, pl.ds(vu16, L)],
                useg_v.at[slot, blk_t, :, :, pl.ds(de, L)], sem)
            pltpu.async_copy(
                iembd_hbm.at[:, :, pl.ds(vi16, L)],
                iseg_v.at[slot, blk_t, :, :, pl.ds(de, L)], sem)
            pltpu.async_copy(
                ubias_hbm.at[pl.ds(vu16, L)], ubias_v.at[pl.ds(dt, L)], sem)
            pltpu.async_copy(
                ibias_hbm.at[pl.ds(vi16, L)], ibias_v.at[pl.ds(dt, L)], sem)

    def drain(slot, sem):
        for blk_t in range(2):
            pltpu.make_async_copy(
                uembd_hbm.at[:, :, pl.ds(0, L * EPB)],
                useg_v.at[slot, blk_t], sem).wait()
            pltpu.make_async_copy(
                iembd_hbm.at[:, :, pl.ds(0, L * EPB)],
                iseg_v.at[slot, blk_t], sem).wait()
        dslot = pl.multiple_of(slot * BIAS_SLOT, L)
        pltpu.make_async_copy(
            ubias_hbm.at[pl.ds(0, BIAS_SLOT)],
            ubias_v.at[pl.ds(dslot, BIAS_SLOT)], sem).wait()
        pltpu.make_async_copy(
            ibias_hbm.at[pl.ds(0, BIAS_SLOT)],
            ibias_v.at[pl.ds(dslot, BIAS_SLOT)], sem).wait()

    def compute(gi, slot):
        j0 = gi * L
        uvec = uidx_v[pl.ds(j0, L)]
        ivec = iidx_v[pl.ds(j0, L)]
        blk = lanes >> 3
        slot_f = jnp.zeros((L,), jnp.int32) + slot
        minor_u = ((lanes & (EPB - 1)) << 4) + (uvec & (L - 1))
        minor_i = ((lanes & (EPB - 1)) << 4) + (ivec & (L - 1))
        uboff = slot * BIAS_SLOT + lanes * L + (uvec & (L - 1))
        iboff = slot * BIAS_SLOT + lanes * L + (ivec & (L - 1))
        acc = (plsc.load_gather(ubias_v, [uboff])
               + plsc.load_gather(ibias_v, [iboff]) + oab)
        for a in range(PLANES):
            af = jnp.full((L,), a, jnp.int32)
            for r in range(SUB):
                rf = jnp.full((L,), r, jnp.int32)
                acc = acc + (
                    plsc.load_gather(useg_v, [slot_f, blk, af, rf, minor_u])
                    * plsc.load_gather(iseg_v, [slot_f, blk, af, rf, minor_i]))
        out_v[pl.ds(j0, L)] = acc

    issue(0, 0, sem0)

    def pipe2(hi, carry):
        g_even = hi * 2
        g_odd = hi * 2 + 1
        issue(g_odd, 1, sem1)
        drain(0, sem0)
        compute(g_even, 0)

        @pl.when(hi < GROUPS // 2 - 1)
        def _():
            issue(g_even + 2, 0, sem0)

        drain(1, sem1)
        compute(g_odd, 1)
        return carry

    lax.fori_loop(0, GROUPS // 2, pipe2, 0)

    pltpu.sync_copy(out_v, out_hbm.at[pl.ds(base, PW)])


def kernel(userIdx, itemIdx, uEmbd, iEmbd, uBias, iBias, overAllBias):
    uidx = userIdx.astype(jnp.int32)
    iidx = itemIdx.astype(jnp.int32)
    ut = uEmbd.T.reshape(PLANES, SUB, uEmbd.shape[0])
    it = iEmbd.T.reshape(PLANES, SUB, iEmbd.shape[0])
    ubias = uBias.reshape(-1)
    ibias = iBias.reshape(-1)
    oab = jnp.broadcast_to(overAllBias.astype(jnp.float32), (L,))
    return _sc_predict(uidx, iidx, ut, it, ubias, ibias, oab)
